# baseline (speedup 1.0000x reference)
import functools
import random as _random

import jax
import jax.numpy as jnp
from jax import lax
from jax.experimental import pallas as pl
from jax.experimental.pallas import tpu as pltpu
from jax.experimental.pallas import tpu_sc as plsc

_R = 8
_NUM_BUCKETS = 50
_E_PARAM = 1


def _selected_buckets(num_buckets):
    rng = _random.Random(0)
    return set(rng.sample(range(num_buckets), min(_E_PARAM, num_buckets)))


def _normalize_body(gate_ref, metric_ref, norm_ref, mn_ref):
    x = metric_ref[0]
    n = norm_ref[0]
    mn_ref[0] = (x / jnp.maximum(n, 1e-12)) * gate_ref[0]


def _normalize_call(gate, metric, norm):
    b, t, f = metric.shape
    return pl.pallas_call(
        _normalize_body,
        grid=(b,),
        in_specs=[
            pl.BlockSpec(memory_space=pltpu.SMEM),
            pl.BlockSpec((1, t, f), lambda i: (i, 0, 0)),
            pl.BlockSpec((1, t, 1), lambda i: (i, 0, 0)),
        ],
        out_specs=pl.BlockSpec((1, t, f), lambda i: (i, 0, 0)),
        out_shape=jax.ShapeDtypeStruct((b, t, f), jnp.float32),
    )(gate, metric, norm)


def _hash_body(mn_ref, proj_ref, key_ref):
    mm = lax.dot_general(proj_ref[...], mn_ref[0], (((1,), (1,)), ((), ())),
                         preferred_element_type=jnp.float32)
    key_ref[0, 0] = jnp.sum((mm > 0).astype(jnp.float32), axis=1)


def _hash_call(mn, projections, hb):
    b, t, f = mn.shape
    h = projections.shape[0]
    return pl.pallas_call(
        _hash_body,
        grid=(b, h // hb),
        in_specs=[
            pl.BlockSpec((1, t, f), lambda i, j: (i, 0, 0)),
            pl.BlockSpec((hb, f), lambda i, j: (j, 0)),
        ],
        out_specs=pl.BlockSpec((1, 1, hb), lambda i, j: (i, 0, j)),
        out_shape=jax.ShapeDtypeStruct((b, 1, h), jnp.float32),
    )(mn, projections)


def _rank_body(ib, t, key_full_ref, key_blk_ref, rank_ref):
    i0 = pl.program_id(1) * ib
    kf = key_full_ref[0, 0, :][None, :]
    kb = key_blk_ref[0, 0, :][:, None]
    jt = lax.broadcasted_iota(jnp.int32, (ib, t), 1)
    it = lax.broadcasted_iota(jnp.int32, (ib, t), 0) + i0
    lt = (kf < kb).astype(jnp.int32)
    eq = ((kf == kb) & (jt < it)).astype(jnp.int32)
    rank_ref[0, 0, :] = jnp.sum(lt + eq, axis=1)


def _rank_call(key, ib):
    b, _, t = key.shape
    return pl.pallas_call(
        functools.partial(_rank_body, ib, t),
        grid=(b, t // ib),
        in_specs=[
            pl.BlockSpec((1, 1, t), lambda i, j: (i, 0, 0)),
            pl.BlockSpec((1, 1, ib), lambda i, j: (i, 0, j)),
        ],
        out_specs=pl.BlockSpec((1, 1, ib), lambda i, j: (i, 0, j)),
        out_shape=jax.ShapeDtypeStruct((b, 1, t), jnp.int32),
    )(key, key)


def _invert_body(ib, t, rank_ref, sidx_ref):
    p0 = pl.program_id(1) * ib
    rj = rank_ref[0, 0, :][None, :]
    pt = lax.broadcasted_iota(jnp.int32, (ib, t), 0) + p0
    tj = lax.broadcasted_iota(jnp.int32, (ib, t), 1)
    sel = (rj == pt)
    sidx_ref[0, 0, :] = jnp.sum(jnp.where(sel, tj, 0), axis=1)


def _invert_call(rank, ib):
    b, _, t = rank.shape
    return pl.pallas_call(
        functools.partial(_invert_body, ib, t),
        grid=(b, t // ib),
        in_specs=[pl.BlockSpec((1, 1, t), lambda i, j: (i, 0, 0))],
        out_specs=pl.BlockSpec((1, 1, ib), lambda i, j: (i, 0, j)),
        out_shape=jax.ShapeDtypeStruct((b, 1, t), jnp.int32),
    )(rank)


def _sc_gather(table, gidx):
    n_rows, d = gidx.shape[0], table.shape[1]
    info = plsc.get_sparse_core_info()
    nw = info.num_cores * info.num_subcores
    per_w = n_rows // nw
    ch = 64
    n_ch = per_w // ch
    nc = info.num_cores
    mesh = plsc.VectorSubcoreMesh(core_axis_name="c", subcore_axis_name="s")

    @functools.partial(
        pl.kernel, mesh=mesh,
        out_type=jax.ShapeDtypeStruct((n_rows, d), jnp.float32),
        scratch_types=[
            pltpu.VMEM((ch,), jnp.int32),
            pltpu.VMEM((ch, d), jnp.float32),
            pltpu.SemaphoreType.DMA,
        ],
    )
    def gk(table_hbm, idx_hbm, out_hbm, idx_v, rows_v, sem):
        wid = lax.axis_index("s") * nc + lax.axis_index("c")
        base = wid * per_w
        for c in range(n_ch):
            off = base + c * ch
            pltpu.sync_copy(idx_hbm.at[pl.ds(off, ch)], idx_v)
            pltpu.async_copy(table_hbm.at[idx_v], rows_v, sem).wait()
            pltpu.sync_copy(rows_v, out_hbm.at[pl.ds(off, ch)])

    return gk(table, gidx)


def _merge_body(bs, f, bucket_ref, mean_ref):
    x = bucket_ref[0]
    n = jnp.sqrt(jnp.sum(x * x, axis=1, keepdims=True))
    nb = x / jnp.maximum(n, 1e-12)
    sim = lax.dot_general(nb, nb, (((1,), (1,)), ((), ())),
                          preferred_element_type=jnp.float32)
    ii = lax.broadcasted_iota(jnp.int32, (bs, bs), 0)
    jj = lax.broadcasted_iota(jnp.int32, (bs, bs), 1)
    scores = jnp.max(jnp.where(jj > ii, sim, 0.0), axis=1, keepdims=True)
    iota_i = lax.broadcasted_iota(jnp.int32, (bs, 1), 0)
    acc = jnp.zeros((1, f), jnp.float32)
    sc = scores
    for _ in range(_R):
        m = jnp.max(sc)
        idxv = jnp.min(jnp.where(sc == m, iota_i, bs))
        sel = (iota_i == idxv)
        acc = acc + jnp.sum(jnp.where(sel, x, 0.0), axis=0, keepdims=True)
        sc = jnp.where(sel, -jnp.inf, sc)
    mean_ref[0] = acc / float(_R)


def _merge_call(bucket):
    b, bs, f = bucket.shape
    return pl.pallas_call(
        functools.partial(_merge_body, bs, f),
        grid=(b,),
        in_specs=[pl.BlockSpec((1, bs, f), lambda i: (i, 0, 0))],
        out_specs=pl.BlockSpec((1, 1, f), lambda i: (i, 0, 0)),
        out_shape=jax.ShapeDtypeStruct((b, 1, f), jnp.float32),
    )(bucket)


def kernel(metric, projections, r):
    b, t, f = metric.shape
    gate = (jnp.minimum(r, t // 2) > 0).astype(jnp.float32).reshape(1)
    norm = jnp.linalg.norm(metric, ord=2, axis=-1, keepdims=True)
    mn = _normalize_call(gate, metric, norm)
    key = _hash_call(mn, projections, hb=1024)
    rank = _rank_call(key, ib=512)
    sidx = _invert_call(rank, ib=512)[:, 0, :]

    bs = (t + _NUM_BUCKETS - 1) // _NUM_BUCKETS
    sel = sorted(_selected_buckets(-(-t // bs)))[0]
    s0 = sel * bs
    tm = t - _R + 1

    offs = (jnp.arange(b, dtype=jnp.int32) * t)[:, None]
    g_sorted = (sidx + offs).reshape(-1)
    merged_idx = jnp.concatenate(
        [sidx[:, :s0], jnp.zeros((b, 1), jnp.int32), sidx[:, s0 + _R:]],
        axis=1)
    g_merged = (merged_idx + offs).reshape(-1)
    total = b * t + b * tm
    padded = -(-total // 256) * 256
    gidx = jnp.concatenate(
        [g_sorted, g_merged,
         jnp.zeros((padded - total,), jnp.int32)])

    allrows = _sc_gather(mn.reshape(b * t, f), gidx)
    sorted_metric = allrows[:b * t].reshape(b, t, f)
    merged_tokens = allrows[b * t:total].reshape(b, tm, f)

    bucket = lax.slice(sorted_metric, (0, s0, 0), (b, s0 + bs, f))
    meanrow = _merge_call(bucket)
    merged_tokens = lax.dynamic_update_slice(merged_tokens, meanrow, (0, s0, 0))
    return merged_tokens, sorted_metric
